# Initial kernel scaffold; baseline (speedup 1.0000x reference)
import jax
import jax.numpy as jnp
from jax.experimental import pallas as pl


def kernel(x, edge_index, edge_attr, batch, params):
    raise NotImplementedError("write your pallas kernel here")


# speedup vs baseline: 3.6427x; 3.6427x over previous
import functools

import jax
import jax.numpy as jnp
from jax import lax
from jax.experimental import pallas as pl
from jax.experimental.pallas import tpu as pltpu
from jax.experimental.pallas import tpu_sc as plsc

N = 50000
E = 800000
D_IN = 9
H = 64
G = 512

NC = 2
NS = 16
C = 128
SUPER = 16
E_PAD = 851968
ACC_ROWS = 51200
WB_ROWS = 3128
OUT_ROWS = NS * WB_ROWS

NB = 2000
NBLK = N // NB


def _conv_sc(table, src2, dstR, attrR, w2, b2, F, split_edges):
    n_chunks = E_PAD // ((NC * NS if split_edges else NS) * C)
    n_super = n_chunks // SUPER
    mesh = plsc.VectorSubcoreMesh(core_axis_name="c", subcore_axis_name="s")
    HF = F // 16

    def body(table_h, src_h, dst_h, attr_h, w_h, b_h, out_h,
             src_v, dst_v, attr_v, rows_v, zbuf_v, w_v, b_v, acc_sh, gsem):
        c = lax.axis_index("c")
        s = lax.axis_index("s")
        pltpu.sync_copy(w_h.at[c], w_v)
        pltpu.sync_copy(b_h.at[c], b_v)
        w_regs = [w_v[pl.ds(h * 16, 16)] for h in range(HF)]
        b_regs = [b_v[pl.ds(h * 16, 16)] for h in range(HF)]

        zero = jnp.zeros((16,), jnp.float32)
        for r in range(C):
            for h in range(HF):
                zbuf_v[r, pl.ds(h * 16, 16)] = zero

        def zchunk(i, _):
            pltpu.sync_copy(zbuf_v, acc_sh.at[pl.ds(s * (ACC_ROWS // NS) + i * C, C)])
            return 0
        lax.fori_loop(0, ACC_ROWS // (NS * C), zchunk, 0)
        plsc.subcore_barrier()

        if split_edges:
            base0 = (c * NS + s) * n_chunks
        else:
            base0 = s * n_chunks

        def super_body(t, _):
            rb = base0 + t * SUPER
            pltpu.sync_copy(src_h.at[c, pl.ds(rb, SUPER)], src_v)
            pltpu.sync_copy(dst_h.at[pl.ds(rb, SUPER)], dst_v)
            pltpu.sync_copy(attr_h.at[pl.ds(rb, SUPER)], attr_v)
            def sub_body(k, _):
                pltpu.async_copy(table_h.at[src_v.at[k]], rows_v, gsem).wait()

                def edge16(e16, _):
                    a16 = attr_v[k, pl.ds(e16 * 16, 16)]
                    for j in range(16):
                        a = a16[j]
                        e = e16 * 16 + j
                        for h in range(HF):
                            v = rows_v[e, pl.ds(h * 16, 16)]
                            rows_v[e, pl.ds(h * 16, 16)] = jnp.maximum(
                                v + a * w_regs[h] + b_regs[h], 0.0)
                    return 0
                lax.fori_loop(0, C // 16, edge16, 0)
                pltpu.sync_copy(rows_v, acc_sh.at[dst_v.at[k]], add=True)
                return 0
            lax.fori_loop(0, SUPER, sub_body, 0)
            return 0
        lax.fori_loop(0, n_super, super_body, 0)
        plsc.subcore_barrier()
        pltpu.sync_copy(acc_sh.at[pl.ds(s * WB_ROWS, WB_ROWS)],
                        out_h.at[pl.ds(c * OUT_ROWS + s * WB_ROWS, WB_ROWS)])

    return pl.kernel(
        body,
        out_type=jax.ShapeDtypeStruct((2 * OUT_ROWS, F), jnp.float32),
        mesh=mesh,
        compiler_params=pltpu.CompilerParams(use_tc_tiling_on_sc=False),
        scratch_types=[
            pltpu.VMEM((SUPER, C), jnp.int32),
            pltpu.VMEM((SUPER, C), jnp.int32),
            pltpu.VMEM((SUPER, C), jnp.float32),
            pltpu.VMEM((C, F), jnp.float32),
            pltpu.VMEM((C, F), jnp.float32),
            pltpu.VMEM((F,), jnp.float32),
            pltpu.VMEM((F,), jnp.float32),
            pltpu.MemorySpace.VMEM_SHARED((ACC_ROWS, F), jnp.float32),
            pltpu.SemaphoreType.DMA,
        ],
    )(table, src2, dstR, attrR, w2, b2)


def _d1a_body(x_ref, p0_ref, p1_ref, w_ref, b_ref, h1_ref, s_ref, q_ref):
    h0 = x_ref[...] + p0_ref[...] + p1_ref[...]
    h1 = jnp.dot(h0, w_ref[...], preferred_element_type=jnp.float32) + b_ref[...]
    h1_ref[...] = h1
    s_ref[...] = jnp.sum(h1, axis=0, keepdims=True).reshape(1, 1, H)
    q_ref[...] = jnp.sum(h1 * h1, axis=0, keepdims=True).reshape(1, 1, H)


def _dense1_a(x_pad, p0, p1, w, b):
    return pl.pallas_call(
        _d1a_body,
        grid=(NBLK,),
        in_specs=[
            pl.BlockSpec((NB, 16), lambda i: (i, 0)),
            pl.BlockSpec((NB, 16), lambda i: (i, 0)),
            pl.BlockSpec((NB, 16), lambda i: (i, 0)),
            pl.BlockSpec((16, H), lambda i: (0, 0)),
            pl.BlockSpec((1, H), lambda i: (0, 0)),
        ],
        out_specs=[
            pl.BlockSpec((NB, H), lambda i: (i, 0)),
            pl.BlockSpec((1, 1, H), lambda i: (i, 0, 0)),
            pl.BlockSpec((1, 1, H), lambda i: (i, 0, 0)),
        ],
        out_shape=[
            jax.ShapeDtypeStruct((N, H), jnp.float32),
            jax.ShapeDtypeStruct((NBLK, 1, H), jnp.float32),
            jax.ShapeDtypeStruct((NBLK, 1, H), jnp.float32),
        ],
    )(x_pad, p0, p1, w, b)


def _d2a_body(h_ref, a0_ref, a1_ref, w_ref, wa_ref, wb_ref, b_ref,
              h1_ref, s_ref, q_ref):
    h1 = (jnp.dot(h_ref[...], w_ref[...], preferred_element_type=jnp.float32)
          + jnp.dot(a0_ref[...], wa_ref[...], preferred_element_type=jnp.float32)
          + jnp.dot(a1_ref[...], wb_ref[...], preferred_element_type=jnp.float32)
          + b_ref[...])
    h1_ref[...] = h1
    s_ref[...] = jnp.sum(h1, axis=0, keepdims=True).reshape(1, 1, H)
    q_ref[...] = jnp.sum(h1 * h1, axis=0, keepdims=True).reshape(1, 1, H)


def _dense2_a(h, a0, a1, w, b):
    return pl.pallas_call(
        _d2a_body,
        grid=(NBLK,),
        in_specs=[
            pl.BlockSpec((NB, H), lambda i: (i, 0)),
            pl.BlockSpec((NB, 32), lambda i: (i, 0)),
            pl.BlockSpec((NB, 32), lambda i: (i, 0)),
            pl.BlockSpec((H, H), lambda i: (0, 0)),
            pl.BlockSpec((32, H), lambda i: (0, 0)),
            pl.BlockSpec((32, H), lambda i: (0, 0)),
            pl.BlockSpec((1, H), lambda i: (0, 0)),
        ],
        out_specs=[
            pl.BlockSpec((NB, H), lambda i: (i, 0)),
            pl.BlockSpec((1, 1, H), lambda i: (i, 0, 0)),
            pl.BlockSpec((1, 1, H), lambda i: (i, 0, 0)),
        ],
        out_shape=[
            jax.ShapeDtypeStruct((N, H), jnp.float32),
            jax.ShapeDtypeStruct((NBLK, 1, H), jnp.float32),
            jax.ShapeDtypeStruct((NBLK, 1, H), jnp.float32),
        ],
    )(h, a0, a1, w, w[:32], w[32:], b)


def _db_body(h1_ref, s_ref, q_ref, g_ref, beta_ref, w_ref, b_ref, o_ref):
    mean = jnp.sum(s_ref[...], axis=0) / N
    ex2 = jnp.sum(q_ref[...], axis=0) / N
    var = ex2 - mean * mean
    inv = lax.rsqrt(var + 1e-5)
    hb = g_ref[...] * (h1_ref[...] - mean) * inv + beta_ref[...]
    hr = jnp.maximum(hb, 0.0)
    o = jnp.dot(hr, w_ref[...], preferred_element_type=jnp.float32) + b_ref[...]
    o_ref[...] = jnp.maximum(o, 0.0)


def _dense_b(h1, s, q, g, beta, w, b):
    return pl.pallas_call(
        _db_body,
        grid=(NBLK,),
        in_specs=[
            pl.BlockSpec((NB, H), lambda i: (i, 0)),
            pl.BlockSpec((NBLK, 1, H), lambda i: (0, 0, 0)),
            pl.BlockSpec((NBLK, 1, H), lambda i: (0, 0, 0)),
            pl.BlockSpec((1, H), lambda i: (0, 0)),
            pl.BlockSpec((1, H), lambda i: (0, 0)),
            pl.BlockSpec((H, H), lambda i: (0, 0)),
            pl.BlockSpec((1, H), lambda i: (0, 0)),
        ],
        out_specs=pl.BlockSpec((NB, H), lambda i: (i, 0)),
        out_shape=jax.ShapeDtypeStruct((N, H), jnp.float32),
    )(h1, s, q, g.reshape(1, H), beta.reshape(1, H), w, b)


def _pool_body(h_ref, bt_ref, wr1_ref, br1_ref, wr2_ref, br2_ref, o_ref,
               acc_ref, cnt_ref):
    i = pl.program_id(0)

    @pl.when(i == 0)
    def _init():
        acc_ref[...] = jnp.zeros_like(acc_ref)
        cnt_ref[...] = jnp.zeros_like(cnt_ref)

    bt = bt_ref[0]
    gid = lax.broadcasted_iota(jnp.int32, (G, NB), 0)
    oh = (gid == bt).astype(jnp.float32)
    acc_ref[...] += jnp.dot(oh, h_ref[...], preferred_element_type=jnp.float32)
    cnt_part = jnp.sum(oh, axis=1, keepdims=True)
    cnt_ref[...] += jnp.broadcast_to(cnt_part, (G, H))

    @pl.when(i == NBLK - 1)
    def _final():
        xg = acc_ref[...] / jnp.maximum(cnt_ref[...], 1.0)
        r = jnp.maximum(
            jnp.dot(xg, wr1_ref[...], preferred_element_type=jnp.float32)
            + br1_ref[...], 0.0)
        o = jnp.dot(r, wr2_ref[...], preferred_element_type=jnp.float32) + br2_ref[...]
        o_ref[...] = 1.0 / (1.0 + jnp.exp(-o))


def _pool_readout(h4, batch3, wr1, br1, wr2, br2):
    return pl.pallas_call(
        _pool_body,
        grid=(NBLK,),
        in_specs=[
            pl.BlockSpec((NB, H), lambda i: (i, 0)),
            pl.BlockSpec((1, 1, NB), lambda i: (i, 0, 0)),
            pl.BlockSpec((H, 32), lambda i: (0, 0)),
            pl.BlockSpec((1, 32), lambda i: (0, 0)),
            pl.BlockSpec((32, 1), lambda i: (0, 0)),
            pl.BlockSpec((1, 1), lambda i: (0, 0)),
        ],
        out_specs=pl.BlockSpec((G, 1), lambda i: (0, 0)),
        out_shape=jax.ShapeDtypeStruct((G, 1), jnp.float32),
        scratch_shapes=[
            pltpu.VMEM((G, H), jnp.float32),
            pltpu.VMEM((G, H), jnp.float32),
        ],
    )(h4, batch3, wr1, br1, wr2, br2)


def kernel(x, edge_index, edge_attr, batch, params):
    p = params
    f32 = jnp.float32
    src = edge_index[0]
    dst = edge_index[1]
    attr = edge_attr[:, 0]

    pad = E_PAD - E
    srcp = jnp.concatenate([src, jnp.zeros((pad,), jnp.int32)])
    dstp = jnp.concatenate([dst, jnp.full((pad,), N, jnp.int32)])
    attrp = jnp.concatenate([attr, jnp.zeros((pad,), f32)])
    nrows = E_PAD // C
    src1 = jnp.stack([srcp, srcp]).reshape(2, nrows, C)
    src2 = jnp.stack([srcp, srcp + N]).reshape(2, nrows, C)
    dstR = dstp.reshape(nrows, C)
    attrR = attrp.reshape(nrows, C)

    x_pad = jnp.pad(x, ((0, 0), (0, 16 - D_IN)))
    w1 = jnp.pad(p["We1"][0], (0, 16 - D_IN))
    b1 = jnp.pad(p["be1"], (0, 16 - D_IN))
    agg1 = _conv_sc(x_pad, src1, dstR, attrR,
                    jnp.stack([w1, w1]), jnp.stack([b1, b1]),
                    F=16, split_edges=True)

    w11p = jnp.pad(p["W11"], ((0, 16 - D_IN), (0, 0)))
    h1, s1, q1 = _dense1_a(x_pad, agg1[:N], agg1[OUT_ROWS:OUT_ROWS + N],
                           w11p, p["b11"].reshape(1, H))
    h2 = _dense_b(h1, s1, q1, p["g1"], p["beta1"], p["W12"],
                  p["b12"].reshape(1, H))

    table2 = jnp.concatenate([h2[:, :32], h2[:, 32:]], axis=0)
    agg2 = _conv_sc(table2, src2, dstR, attrR,
                    p["We2"][0].reshape(2, 32), p["be2"].reshape(2, 32),
                    F=32, split_edges=False)

    h3, s2, q2 = _dense2_a(h2, agg2[:N], agg2[OUT_ROWS:OUT_ROWS + N],
                           p["W21"], p["b21"].reshape(1, H))
    h4 = _dense_b(h3, s2, q2, p["g2"], p["beta2"], p["W22"],
                  p["b22"].reshape(1, H))

    batch3 = batch.reshape(NBLK, 1, NB)
    out = _pool_readout(h4, batch3, p["Wr1"], p["br1"].reshape(1, 32),
                        p["Wr2"], p["br2"].reshape(1, 1))
    return out


# speedup vs baseline: 4.3369x; 1.1906x over previous
import functools

import jax
import jax.numpy as jnp
from jax import lax
from jax.experimental import pallas as pl
from jax.experimental.pallas import tpu as pltpu
from jax.experimental.pallas import tpu_sc as plsc

N = 50000
E = 800000
D_IN = 9
H = 64
G = 512

NC = 2
NS = 16
C = 128
SUPER = 8
E_PAD = 851968
ACC_ROWS = 51200
WB_ROWS = 3128
OUT_ROWS = NS * WB_ROWS

NB = 2000
NBLK = N // NB


def _conv_sc(table, src2, dstR, attrR, w2, b2, F, split_edges):
    n_chunks = E_PAD // ((NC * NS if split_edges else NS) * C)
    n_super = n_chunks // SUPER
    mesh = plsc.VectorSubcoreMesh(core_axis_name="c", subcore_axis_name="s")
    HF = F // 16

    def body(table_h, src_h, dst_h, attr_h, w_h, b_h, out_h,
             src_v, dst_v, attr_v,
             rb0, rb1, rb2, rb3, mb0, mb1,
             w_v, b_v, acc_sh,
             g0, g1, g2, g3, s0, s1, isem):
        rbufs = [rb0, rb1, rb2, rb3]
        mbufs = [mb0, mb1]
        gsems = [g0, g1, g2, g3]
        ssems = [s0, s1]
        c = lax.axis_index("c")
        s = lax.axis_index("s")
        pltpu.sync_copy(w_h.at[c], w_v)
        pltpu.sync_copy(b_h.at[c], b_v)
        w_regs = [w_v[pl.ds(h * 16, 16)] for h in range(HF)]
        b_regs = [b_v[pl.ds(h * 16, 16)] for h in range(HF)]

        zero = jnp.zeros((16,), jnp.float32)
        for r in range(C):
            for h in range(HF):
                rb0[r, pl.ds(h * 16, 16)] = zero

        def zchunk(i, _):
            pltpu.sync_copy(rb0, acc_sh.at[pl.ds(s * (ACC_ROWS // NS) + i * C, C)])
            return 0
        lax.fori_loop(0, ACC_ROWS // (NS * C), zchunk, 0)
        plsc.subcore_barrier()

        if split_edges:
            base0 = (c * NS + s) * n_chunks
        else:
            base0 = s * n_chunks

        def gather(krow, bi):
            pltpu.async_copy(table_h.at[src_v.at[krow]], rbufs[bi], gsems[bi])

        def compute(krow, rb, mb):
            @plsc.parallel_loop(0, C // 16, unroll=1)
            def _edges(e16):
                a16 = attr_v[krow, pl.ds(e16 * 16, 16)]
                for j in range(16):
                    a = a16[j]
                    e = e16 * 16 + j
                    for h in range(HF):
                        v = rb[e, pl.ds(h * 16, 16)]
                        mb[e, pl.ds(h * 16, 16)] = jnp.maximum(
                            v + a * w_regs[h] + b_regs[h], 0.0)

        def super_body(t, _):
            row0 = base0 + t * SUPER
            i1 = pltpu.async_copy(src_h.at[c, pl.ds(row0, SUPER)], src_v, isem)
            i2 = pltpu.async_copy(dst_h.at[pl.ds(row0, SUPER)], dst_v, isem)
            i3 = pltpu.async_copy(attr_h.at[pl.ds(row0, SUPER)], attr_v, isem)
            i1.wait()
            i2.wait()
            i3.wait()
            for pre in range(3):
                gather(pre, pre)

            def quad(k4, _):
                for j in range(4):
                    krow = k4 * 4 + j
                    mj = j % 2
                    gb = (j + 3) % 4
                    if j == 0:
                        gather(krow + 3, gb)
                    else:
                        @pl.when(k4 < SUPER // 4 - 1)
                        def _pref():
                            gather(krow + 3, gb)
                    pltpu.make_async_copy(
                        table_h.at[src_v.at[krow]], rbufs[j], gsems[j]).wait()
                    if j >= 2:
                        pltpu.make_async_copy(
                            mbufs[mj], acc_sh.at[dst_v.at[krow]], ssems[mj]).wait()
                    else:
                        @pl.when((t > 0) | (k4 >= 1))
                        def _wait_scat():
                            pltpu.make_async_copy(
                                mbufs[mj], acc_sh.at[dst_v.at[krow]], ssems[mj]).wait()
                    compute(krow, rbufs[j], mbufs[mj])
                    pltpu.async_copy(
                        mbufs[mj], acc_sh.at[dst_v.at[krow]], ssems[mj], add=True)
                return 0
            lax.fori_loop(0, SUPER // 4, quad, 0)
            return 0
        lax.fori_loop(0, n_super, super_body, 0)
        for j in range(2):
            pltpu.make_async_copy(
                mbufs[j], acc_sh.at[dst_v.at[SUPER - 2 + j]], ssems[j]).wait()
        plsc.subcore_barrier()
        pltpu.sync_copy(acc_sh.at[pl.ds(s * WB_ROWS, WB_ROWS)],
                        out_h.at[pl.ds(c * OUT_ROWS + s * WB_ROWS, WB_ROWS)])

    return pl.kernel(
        body,
        out_type=jax.ShapeDtypeStruct((2 * OUT_ROWS, F), jnp.float32),
        mesh=mesh,
        compiler_params=pltpu.CompilerParams(use_tc_tiling_on_sc=False),
        scratch_types=(
            [pltpu.VMEM((SUPER, C), jnp.int32),
             pltpu.VMEM((SUPER, C), jnp.int32),
             pltpu.VMEM((SUPER, C), jnp.float32)]
            + [pltpu.VMEM((C, F), jnp.float32)] * 6
            + [pltpu.VMEM((F,), jnp.float32)] * 2
            + [pltpu.MemorySpace.VMEM_SHARED((ACC_ROWS, F), jnp.float32)]
            + [pltpu.SemaphoreType.DMA] * 7
        ),
    )(table, src2, dstR, attrR, w2, b2)


def _d1a_body(x_ref, p0_ref, p1_ref, w_ref, b_ref, h1_ref, s_ref, q_ref):
    h0 = x_ref[...] + p0_ref[...] + p1_ref[...]
    h1 = jnp.dot(h0, w_ref[...], preferred_element_type=jnp.float32) + b_ref[...]
    h1_ref[...] = h1
    s_ref[...] = jnp.sum(h1, axis=0, keepdims=True).reshape(1, 1, H)
    q_ref[...] = jnp.sum(h1 * h1, axis=0, keepdims=True).reshape(1, 1, H)


def _dense1_a(x_pad, p0, p1, w, b):
    return pl.pallas_call(
        _d1a_body,
        grid=(NBLK,),
        in_specs=[
            pl.BlockSpec((NB, 16), lambda i: (i, 0)),
            pl.BlockSpec((NB, 16), lambda i: (i, 0)),
            pl.BlockSpec((NB, 16), lambda i: (i, 0)),
            pl.BlockSpec((16, H), lambda i: (0, 0)),
            pl.BlockSpec((1, H), lambda i: (0, 0)),
        ],
        out_specs=[
            pl.BlockSpec((NB, H), lambda i: (i, 0)),
            pl.BlockSpec((1, 1, H), lambda i: (i, 0, 0)),
            pl.BlockSpec((1, 1, H), lambda i: (i, 0, 0)),
        ],
        out_shape=[
            jax.ShapeDtypeStruct((N, H), jnp.float32),
            jax.ShapeDtypeStruct((NBLK, 1, H), jnp.float32),
            jax.ShapeDtypeStruct((NBLK, 1, H), jnp.float32),
        ],
    )(x_pad, p0, p1, w, b)


def _d2a_body(h_ref, a0_ref, a1_ref, w_ref, wa_ref, wb_ref, b_ref,
              h1_ref, s_ref, q_ref):
    h1 = (jnp.dot(h_ref[...], w_ref[...], preferred_element_type=jnp.float32)
          + jnp.dot(a0_ref[...], wa_ref[...], preferred_element_type=jnp.float32)
          + jnp.dot(a1_ref[...], wb_ref[...], preferred_element_type=jnp.float32)
          + b_ref[...])
    h1_ref[...] = h1
    s_ref[...] = jnp.sum(h1, axis=0, keepdims=True).reshape(1, 1, H)
    q_ref[...] = jnp.sum(h1 * h1, axis=0, keepdims=True).reshape(1, 1, H)


def _dense2_a(h, a0, a1, w, b):
    return pl.pallas_call(
        _d2a_body,
        grid=(NBLK,),
        in_specs=[
            pl.BlockSpec((NB, H), lambda i: (i, 0)),
            pl.BlockSpec((NB, 32), lambda i: (i, 0)),
            pl.BlockSpec((NB, 32), lambda i: (i, 0)),
            pl.BlockSpec((H, H), lambda i: (0, 0)),
            pl.BlockSpec((32, H), lambda i: (0, 0)),
            pl.BlockSpec((32, H), lambda i: (0, 0)),
            pl.BlockSpec((1, H), lambda i: (0, 0)),
        ],
        out_specs=[
            pl.BlockSpec((NB, H), lambda i: (i, 0)),
            pl.BlockSpec((1, 1, H), lambda i: (i, 0, 0)),
            pl.BlockSpec((1, 1, H), lambda i: (i, 0, 0)),
        ],
        out_shape=[
            jax.ShapeDtypeStruct((N, H), jnp.float32),
            jax.ShapeDtypeStruct((NBLK, 1, H), jnp.float32),
            jax.ShapeDtypeStruct((NBLK, 1, H), jnp.float32),
        ],
    )(h, a0, a1, w, w[:32], w[32:], b)


def _db_body(h1_ref, s_ref, q_ref, g_ref, beta_ref, w_ref, b_ref, o_ref):
    mean = jnp.sum(s_ref[...], axis=0) / N
    ex2 = jnp.sum(q_ref[...], axis=0) / N
    var = ex2 - mean * mean
    inv = lax.rsqrt(var + 1e-5)
    hb = g_ref[...] * (h1_ref[...] - mean) * inv + beta_ref[...]
    hr = jnp.maximum(hb, 0.0)
    o = jnp.dot(hr, w_ref[...], preferred_element_type=jnp.float32) + b_ref[...]
    o_ref[...] = jnp.maximum(o, 0.0)


def _dense_b(h1, s, q, g, beta, w, b):
    return pl.pallas_call(
        _db_body,
        grid=(NBLK,),
        in_specs=[
            pl.BlockSpec((NB, H), lambda i: (i, 0)),
            pl.BlockSpec((NBLK, 1, H), lambda i: (0, 0, 0)),
            pl.BlockSpec((NBLK, 1, H), lambda i: (0, 0, 0)),
            pl.BlockSpec((1, H), lambda i: (0, 0)),
            pl.BlockSpec((1, H), lambda i: (0, 0)),
            pl.BlockSpec((H, H), lambda i: (0, 0)),
            pl.BlockSpec((1, H), lambda i: (0, 0)),
        ],
        out_specs=pl.BlockSpec((NB, H), lambda i: (i, 0)),
        out_shape=jax.ShapeDtypeStruct((N, H), jnp.float32),
    )(h1, s, q, g.reshape(1, H), beta.reshape(1, H), w, b)


def _pool_body(h_ref, bt_ref, wr1_ref, br1_ref, wr2_ref, br2_ref, o_ref,
               acc_ref, cnt_ref):
    i = pl.program_id(0)

    @pl.when(i == 0)
    def _init():
        acc_ref[...] = jnp.zeros_like(acc_ref)
        cnt_ref[...] = jnp.zeros_like(cnt_ref)

    bt = bt_ref[0]
    gid = lax.broadcasted_iota(jnp.int32, (G, NB), 0)
    oh = (gid == bt).astype(jnp.float32)
    acc_ref[...] += jnp.dot(oh, h_ref[...], preferred_element_type=jnp.float32)
    cnt_part = jnp.sum(oh, axis=1, keepdims=True)
    cnt_ref[...] += jnp.broadcast_to(cnt_part, (G, H))

    @pl.when(i == NBLK - 1)
    def _final():
        xg = acc_ref[...] / jnp.maximum(cnt_ref[...], 1.0)
        r = jnp.maximum(
            jnp.dot(xg, wr1_ref[...], preferred_element_type=jnp.float32)
            + br1_ref[...], 0.0)
        o = jnp.dot(r, wr2_ref[...], preferred_element_type=jnp.float32) + br2_ref[...]
        o_ref[...] = 1.0 / (1.0 + jnp.exp(-o))


def _pool_readout(h4, batch3, wr1, br1, wr2, br2):
    return pl.pallas_call(
        _pool_body,
        grid=(NBLK,),
        in_specs=[
            pl.BlockSpec((NB, H), lambda i: (i, 0)),
            pl.BlockSpec((1, 1, NB), lambda i: (i, 0, 0)),
            pl.BlockSpec((H, 32), lambda i: (0, 0)),
            pl.BlockSpec((1, 32), lambda i: (0, 0)),
            pl.BlockSpec((32, 1), lambda i: (0, 0)),
            pl.BlockSpec((1, 1), lambda i: (0, 0)),
        ],
        out_specs=pl.BlockSpec((G, 1), lambda i: (0, 0)),
        out_shape=jax.ShapeDtypeStruct((G, 1), jnp.float32),
        scratch_shapes=[
            pltpu.VMEM((G, H), jnp.float32),
            pltpu.VMEM((G, H), jnp.float32),
        ],
    )(h4, batch3, wr1, br1, wr2, br2)


def kernel(x, edge_index, edge_attr, batch, params):
    p = params
    f32 = jnp.float32
    src = edge_index[0]
    dst = edge_index[1]
    attr = edge_attr[:, 0]

    pad = E_PAD - E
    srcp = jnp.concatenate([src, jnp.zeros((pad,), jnp.int32)])
    dstp = jnp.concatenate([dst, jnp.full((pad,), N, jnp.int32)])
    attrp = jnp.concatenate([attr, jnp.zeros((pad,), f32)])
    nrows = E_PAD // C
    src1 = jnp.stack([srcp, srcp]).reshape(2, nrows, C)
    src2 = jnp.stack([srcp, srcp + N]).reshape(2, nrows, C)
    dstR = dstp.reshape(nrows, C)
    attrR = attrp.reshape(nrows, C)

    x_pad = jnp.pad(x, ((0, 0), (0, 16 - D_IN)))
    w1 = jnp.pad(p["We1"][0], (0, 16 - D_IN))
    b1 = jnp.pad(p["be1"], (0, 16 - D_IN))
    agg1 = _conv_sc(x_pad, src1, dstR, attrR,
                    jnp.stack([w1, w1]), jnp.stack([b1, b1]),
                    F=16, split_edges=True)

    w11p = jnp.pad(p["W11"], ((0, 16 - D_IN), (0, 0)))
    h1, s1, q1 = _dense1_a(x_pad, agg1[:N], agg1[OUT_ROWS:OUT_ROWS + N],
                           w11p, p["b11"].reshape(1, H))
    h2 = _dense_b(h1, s1, q1, p["g1"], p["beta1"], p["W12"],
                  p["b12"].reshape(1, H))

    table2 = jnp.concatenate([h2[:, :32], h2[:, 32:]], axis=0)
    agg2 = _conv_sc(table2, src2, dstR, attrR,
                    p["We2"][0].reshape(2, 32), p["be2"].reshape(2, 32),
                    F=32, split_edges=False)

    h3, s2, q2 = _dense2_a(h2, agg2[:N], agg2[OUT_ROWS:OUT_ROWS + N],
                           p["W21"], p["b21"].reshape(1, H))
    h4 = _dense_b(h3, s2, q2, p["g2"], p["beta2"], p["W22"],
                  p["b22"].reshape(1, H))

    batch3 = batch.reshape(NBLK, 1, NB)
    out = _pool_readout(h4, batch3, p["Wr1"], p["br1"].reshape(1, 32),
                        p["Wr2"], p["br2"].reshape(1, 1))
    return out
